# baseline (speedup 1.0000x reference)
import functools

import jax
import jax.numpy as jnp
import numpy as np
from jax import lax
from jax.experimental import pallas as pl
from jax.experimental.pallas import tpu as pltpu
from jax.experimental.pallas import tpu_sc as plsc

_BATCH = 4096
_HIST = 50
_HISTP = 56
_D = 256
_NC = 2
_NS = 16
_NW = _NC * _NS
_BPW = _BATCH // _NW
_NBUF = 4

_POS = np.concatenate([np.arange(_HIST), np.full(_HISTP - _HIST, _HIST - 1)])

_mesh = plsc.VectorSubcoreMesh(core_axis_name="c", subcore_axis_name="s")


@functools.partial(
    pl.kernel,
    mesh=_mesh,
    out_type=jax.ShapeDtypeStruct((_BATCH, _HISTP, _D), jnp.float32),
    scratch_types=[
        pltpu.VMEM((_BPW, _HISTP), jnp.int32),
        pltpu.VMEM((_NBUF, _HISTP, _D), jnp.float32),
        pltpu.SemaphoreType.DMA,
        pltpu.SemaphoreType.DMA,
    ],
)
def _gather_all(idx_hbm, table_hbm, out_hbm, idx_v, rows_v, gsem, ssem):
    wid = lax.axis_index("s") * _NC + lax.axis_index("c")
    wb = wid * _BPW
    pltpu.sync_copy(idx_hbm.at[pl.ds(wb, _BPW)], idx_v)

    def gather_copy(c, b):
        return pltpu.make_async_copy(
            table_hbm.at[idx_v.at[c]], rows_v.at[b], gsem
        )

    def store_copy(c, b):
        return pltpu.make_async_copy(rows_v.at[b], out_hbm.at[wb + c], ssem)

    gather_copy(0, 0).start()
    gather_copy(1, 1).start()

    def body(c, carry):
        b = lax.rem(c, _NBUF)
        gather_copy(c, b).wait()

        @pl.when(c >= 2)
        def _():
            store_copy(c - 2, lax.rem(c + 2, _NBUF)).wait()

        @pl.when(c + 2 < _BPW)
        def _():
            gather_copy(c + 2, lax.rem(c + 2, _NBUF)).start()

        store_copy(c, b).start()
        return carry

    lax.fori_loop(0, _BPW, body, 0)
    store_copy(_BPW - 2, (_BPW - 2) % _NBUF).wait()
    store_copy(_BPW - 1, (_BPW - 1) % _NBUF).wait()


_BB = 32


def _relayout_body(i_ref, o_ref):
    o_ref[...] = i_ref[:, : _HIST, :]


def _tc_relayout(mid):
    return pl.pallas_call(
        _relayout_body,
        grid=(_BATCH // _BB,),
        in_specs=[pl.BlockSpec((_BB, _HISTP, _D), lambda j: (j, 0, 0))],
        out_specs=pl.BlockSpec((_BB, _HIST, _D), lambda j: (j, 0, 0)),
        out_shape=jax.ShapeDtypeStruct((_BATCH, _HIST, _D), jnp.float32),
    )(mid)


def kernel(indices, table):
    idxp = indices.astype(jnp.int32)[:, _POS]
    return _tc_relayout(_gather_all(idxp, table))


# speedup vs baseline: 3.1472x; 3.1472x over previous
import functools

import jax
import jax.numpy as jnp
from jax import lax
from jax.experimental import pallas as pl
from jax.experimental.pallas import tpu as pltpu
from jax.experimental.pallas import tpu_sc as plsc

_BATCH = 4096
_HIST = 50
_D = 256
_NC = 2
_NS = 16
_NW = _NC * _NS
_BPW = _BATCH // _NW
_NBUF = 3

_mesh = plsc.VectorSubcoreMesh(core_axis_name="c", subcore_axis_name="s")


@functools.partial(
    pl.kernel,
    mesh=_mesh,
    out_type=jax.ShapeDtypeStruct((_HIST, _BATCH, _D), jnp.float32),
    scratch_types=[
        pltpu.VMEM((_HIST, _BPW), jnp.int32),
        pltpu.VMEM((_NBUF, _BPW, _D), jnp.float32),
        pltpu.SemaphoreType.DMA,
        pltpu.SemaphoreType.DMA,
    ],
)
def _gather_all(idx_hbm, table_hbm, out_hbm, idx_v, rows_v, gsem, ssem):
    wid = lax.axis_index("s") * _NC + lax.axis_index("c")
    wb = wid * _BPW
    pltpu.sync_copy(idx_hbm.at[:, pl.ds(wb, _BPW)], idx_v)

    def gather_copy(h, b):
        return pltpu.make_async_copy(
            table_hbm.at[idx_v.at[h]], rows_v.at[b], gsem
        )

    def store_copy(h, b):
        return pltpu.make_async_copy(
            rows_v.at[b], out_hbm.at[h, pl.ds(wb, _BPW)], ssem
        )

    gather_copy(0, 0).start()
    gather_copy(1, 1).start()

    def body(h, carry):
        b = lax.rem(h, _NBUF)
        gather_copy(h, b).wait()

        @pl.when(h >= 1)
        def _():
            store_copy(h - 1, lax.rem(h + 2, _NBUF)).wait()

        @pl.when(h + 2 < _HIST)
        def _():
            gather_copy(h + 2, lax.rem(h + 2, _NBUF)).start()

        store_copy(h, b).start()
        return carry

    lax.fori_loop(0, _HIST, body, 0)
    store_copy(_HIST - 1, (_HIST - 1) % _NBUF).wait()


def kernel(indices, table):
    idx_t = indices.astype(jnp.int32).T
    mid = _gather_all(idx_t, table)
    return jnp.transpose(mid, (1, 0, 2))


# speedup vs baseline: 3.1498x; 1.0008x over previous
import functools

import jax
import jax.numpy as jnp
from jax import lax
from jax.experimental import pallas as pl
from jax.experimental.pallas import tpu as pltpu
from jax.experimental.pallas import tpu_sc as plsc

_BATCH = 4096
_HIST = 50
_D = 256
_NC = 2
_NS = 16
_NW = _NC * _NS
_BPW = _BATCH // _NW
_NBUF = 3

_mesh = plsc.VectorSubcoreMesh(core_axis_name="c", subcore_axis_name="s")


@functools.partial(
    pl.kernel,
    mesh=_mesh,
    out_type=jax.ShapeDtypeStruct((_HIST, _BATCH, _D), jnp.float32),
    scratch_types=[
        pltpu.VMEM((_HIST, _BPW), jnp.int32),
        pltpu.VMEM((_NBUF, _BPW, _D), jnp.float32),
        pltpu.SemaphoreType.DMA,
        pltpu.SemaphoreType.DMA,
    ],
)
def _gather_all(idx_hbm, table_hbm, out_hbm, idx_v, rows_v, gsem, ssem):
    wid = lax.axis_index("s") * _NC + lax.axis_index("c")
    wb = wid * _BPW
    pltpu.sync_copy(idx_hbm.at[:, pl.ds(wb, _BPW)], idx_v)

    def gather_copies(h, b):
        return [
            pltpu.make_async_copy(
                table_hbm.at[idx_v.at[h, pl.ds(j * 64, 64)]],
                rows_v.at[b, pl.ds(j * 64, 64)],
                gsem,
            )
            for j in range(2)
        ]

    def store_copy(h, b):
        return pltpu.make_async_copy(
            rows_v.at[b], out_hbm.at[h, pl.ds(wb, _BPW)], ssem
        )

    for cp in gather_copies(0, 0) + gather_copies(1, 1):
        cp.start()

    def body(h, carry):
        b = lax.rem(h, _NBUF)
        for cp in gather_copies(h, b):
            cp.wait()

        @pl.when(h >= 1)
        def _():
            store_copy(h - 1, lax.rem(h + 2, _NBUF)).wait()

        @pl.when(h + 2 < _HIST)
        def _():
            for cp in gather_copies(h + 2, lax.rem(h + 2, _NBUF)):
                cp.start()

        store_copy(h, b).start()
        return carry

    lax.fori_loop(0, _HIST, body, 0)
    store_copy(_HIST - 1, (_HIST - 1) % _NBUF).wait()


def kernel(indices, table):
    idx_t = indices.astype(jnp.int32).T
    mid = _gather_all(idx_t, table)
    return jnp.transpose(mid, (1, 0, 2))
